# baseline (speedup 1.0000x reference)
import functools

import jax
import jax.numpy as jnp
import numpy as np
from jax.experimental import pallas as pl
from jax.experimental.pallas import tpu as pltpu

F = 26
V = 100000
E = 16
T = 16
B = 4096
PF = F * F

BT = 32


P = F * (F - 1) // 2


def _afm_body(fm_ref, lin_ref, wc_ref, hb_ref, bb_ref, out_ref):
    bt = fm_ref.shape[0]
    fm = fm_ref[...].reshape(bt, F, E)
    prod = (fm[:, :, None, :] * fm[:, None, :, :]).reshape(bt * PF, E)
    zq = jnp.dot(prod, wc_ref[...], preferred_element_type=jnp.float32)
    att = jax.nn.relu(zq[:, :T] + bb_ref[...])
    q = zq[:, T:T + 1].reshape(bt, PF, 1)
    s = jnp.dot(att, hb_ref[...],
                preferred_element_type=jnp.float32)[:, 0:1].reshape(bt, PF, 1)
    p_id = jax.lax.broadcasted_iota(jnp.int32, (bt, PF, 1), 1)
    mask = (p_id % F) > (p_id // F)
    s = jnp.where(mask, s, -1e30)
    m = jnp.max(s, axis=1, keepdims=True)
    e = jnp.where(mask, jnp.exp(s - m), 0.0)
    denom = jnp.sum(e, axis=1)
    num = jnp.sum(e * q, axis=1)
    afm = num / denom
    lin_sum = jnp.sum(lin_ref[...], axis=1, keepdims=True)
    out_ref[...] = jax.nn.sigmoid(afm + lin_sum)


def _afm_stage(fm_flat, lin, wc, hb, bb):
    return pl.pallas_call(
        _afm_body,
        grid=(B // BT,),
        in_specs=[
            pl.BlockSpec((BT, F * E), lambda i: (i, 0)),
            pl.BlockSpec((BT, F), lambda i: (i, 0)),
            pl.BlockSpec((E, 32), lambda i: (0, 0)),
            pl.BlockSpec((T, 8), lambda i: (0, 0)),
            pl.BlockSpec((1, T), lambda i: (0, 0)),
        ],
        out_specs=pl.BlockSpec((BT, 1), lambda i: (i, 0)),
        out_shape=jax.ShapeDtypeStruct((B, 1), jnp.float32),
    )(fm_flat, lin, wc, hb, bb)


def kernel(indices, fm_table, linear_table, att_W, att_b, att_h, proj_p,
           training):
    del training
    offsets = (jnp.arange(F, dtype=indices.dtype) * (V + 1))[None, :]
    idx = (indices + offsets).astype(jnp.int32)

    fm_rows = jnp.take(fm_table, idx.reshape(-1), axis=0)
    lin_vals = jnp.take(linear_table, idx.reshape(-1), axis=0)

    wc = jnp.zeros((E, 32), jnp.float32)
    wc = wc.at[:, :T].set(att_W).at[:, T].set(proj_p[:, 0])
    hb = jnp.zeros((T, 8), jnp.float32).at[:, 0].set(att_h[:, 0])
    bb = att_b[None, :]

    fm_flat = fm_rows.reshape(B, F * E)
    lin = lin_vals.reshape(B, F)
    return _afm_stage(fm_flat, lin, wc, hb, bb)


# speedup vs baseline: 2.5595x; 2.5595x over previous
import jax
import jax.numpy as jnp
from jax.experimental import pallas as pl

F = 26
V = 100000
E = 16
T = 16
B = 4096
PF = F * F
G = 16
NCHUNK = (PF + G - 1) // G
PPAD = NCHUNK * G
LPAD = PPAD * E - PF * E

BT = 64


def _afm_body(fm_ref, lin_ref, wbig_ref, hbig_ref, bias_ref, out_ref):
    bt = fm_ref.shape[0]
    fm = fm_ref[...]
    wbig = wbig_ref[...]
    hbig = hbig_ref[...]
    bias = bias_ref[...]

    rep = jnp.concatenate(
        [jnp.tile(fm[:, i * E:(i + 1) * E], (1, F)) for i in range(F)], axis=1)
    tiled = jnp.tile(fm, (1, F))
    prod = rep * tiled
    prod = jnp.concatenate(
        [prod, jnp.zeros((bt, LPAD), jnp.float32)], axis=1)

    lane32 = jax.lax.broadcasted_iota(jnp.int32, (bt, G * 32), 1) % 32
    s_parts, q_parts = [], []
    for c in range(NCHUNK):
        pc = prod[:, c * G * E:(c + 1) * G * E]
        zc = jnp.dot(pc, wbig, preferred_element_type=jnp.float32)
        ac = jnp.where(lane32 < T, jax.nn.relu(zc + bias), zc)
        sq = jnp.dot(ac, hbig, preferred_element_type=jnp.float32)
        s_parts.append(sq[:, :G])
        q_parts.append(sq[:, G:])
    s = jnp.concatenate(s_parts, axis=1)
    q = jnp.concatenate(q_parts, axis=1)

    p_id = jax.lax.broadcasted_iota(jnp.int32, (bt, PPAD), 1)
    mask = ((p_id % F) > (p_id // F)) & (p_id < PF)
    s = jnp.where(mask, s, -1e30)
    m = jnp.max(s, axis=1, keepdims=True)
    e = jnp.where(mask, jnp.exp(s - m), 0.0)
    denom = jnp.sum(e, axis=1, keepdims=True)
    num = jnp.sum(e * q, axis=1, keepdims=True)
    afm = num / denom
    lin_sum = jnp.sum(lin_ref[...], axis=1, keepdims=True)
    out_ref[...] = jax.nn.sigmoid(afm + lin_sum)


def _afm_stage(fm_flat, lin, wbig, hbig, bias):
    return pl.pallas_call(
        _afm_body,
        grid=(B // BT,),
        in_specs=[
            pl.BlockSpec((BT, F * E), lambda i: (i, 0)),
            pl.BlockSpec((BT, F), lambda i: (i, 0)),
            pl.BlockSpec((G * E, G * 32), lambda i: (0, 0)),
            pl.BlockSpec((G * 32, 2 * G), lambda i: (0, 0)),
            pl.BlockSpec((1, G * 32), lambda i: (0, 0)),
        ],
        out_specs=pl.BlockSpec((BT, 1), lambda i: (i, 0)),
        out_shape=jax.ShapeDtypeStruct((B, 1), jnp.float32),
    )(fm_flat, lin, wbig, hbig, bias)


def kernel(indices, fm_table, linear_table, att_W, att_b, att_h, proj_p,
           training):
    del training
    offsets = (jnp.arange(F, dtype=indices.dtype) * (V + 1))[None, :]
    idx = (indices + offsets).astype(jnp.int32)

    fm_rows = jnp.take(fm_table, idx.reshape(-1), axis=0)
    lin_vals = jnp.take(linear_table, idx.reshape(-1), axis=0)

    wc = jnp.zeros((E, 32), jnp.float32)
    wc = wc.at[:, :T].set(att_W).at[:, T].set(proj_p[:, 0])
    eye = jnp.eye(G, dtype=jnp.float32)
    wbig = jnp.kron(eye, wc)
    hpad = jnp.zeros((32, 1), jnp.float32).at[:T, 0].set(att_h[:, 0])
    e16 = jnp.zeros((32, 1), jnp.float32).at[T, 0].set(1.0)
    hbig = jnp.concatenate(
        [jnp.kron(eye, hpad), jnp.kron(eye, e16)], axis=1)
    bias = jnp.tile(
        jnp.concatenate([att_b, jnp.zeros((T,), jnp.float32)])[None, :],
        (1, G))

    fm_flat = fm_rows.reshape(B, F * E)
    lin = lin_vals.reshape(B, F)
    return _afm_stage(fm_flat, lin, wbig, hbig, bias)


# speedup vs baseline: 5.4232x; 2.1189x over previous
import jax
import jax.numpy as jnp
from jax.experimental import pallas as pl

F = 26
V = 100000
E = 16
T = 16
B = 4096
P = F * (F - 1) // 2
G = 16
NCHUNK = (P + G - 1) // G
PPAD = NCHUNK * G
LPAD = (PPAD - P) * E

BT = 256


def _afm_body(fm_ref, lin_ref, wbig_ref, hbig_ref, qbig_ref, bias_ref,
              out_ref):
    bt = fm_ref.shape[0]
    fm = fm_ref[...]
    wbig = wbig_ref[...]
    hbig = hbig_ref[...]
    qbig = qbig_ref[...]
    bias = bias_ref[...]

    reps = [jnp.tile(fm[:, i * E:(i + 1) * E], (1, F - 1 - i))
            for i in range(F - 1)]
    rest = [fm[:, (i + 1) * E:] for i in range(F - 1)]
    prod = jnp.concatenate(reps, axis=1) * jnp.concatenate(rest, axis=1)
    prod = jnp.concatenate(
        [prod, jnp.zeros((bt, LPAD), jnp.float32)], axis=1)

    s_parts, q_parts = [], []
    for c in range(NCHUNK):
        pc = prod[:, c * G * E:(c + 1) * G * E]
        zc = jnp.dot(pc, wbig, preferred_element_type=jnp.float32)
        ac = jax.nn.relu(zc + bias)
        s_parts.append(jnp.dot(ac, hbig, preferred_element_type=jnp.float32))
        q_parts.append(jnp.dot(pc, qbig, preferred_element_type=jnp.float32))
    s = jnp.concatenate(s_parts, axis=1)
    q = jnp.concatenate(q_parts, axis=1)

    p_id = jax.lax.broadcasted_iota(jnp.int32, (bt, PPAD), 1)
    mask = p_id < P
    s = jnp.where(mask, s, -1e30)
    m = jnp.max(s, axis=1, keepdims=True)
    e = jnp.where(mask, jnp.exp(s - m), 0.0)
    denom = jnp.sum(e, axis=1, keepdims=True)
    num = jnp.sum(e * q, axis=1, keepdims=True)
    afm = num / denom
    lin_sum = jnp.sum(lin_ref[...], axis=1, keepdims=True)
    out_ref[...] = jax.nn.sigmoid(afm + lin_sum)


def _afm_stage(fm_flat, lin, wbig, hbig, qbig, bias):
    return pl.pallas_call(
        _afm_body,
        grid=(B // BT,),
        in_specs=[
            pl.BlockSpec((BT, F * E), lambda i: (i, 0)),
            pl.BlockSpec((BT, F), lambda i: (i, 0)),
            pl.BlockSpec((G * E, G * 32), lambda i: (0, 0)),
            pl.BlockSpec((G * 32, G), lambda i: (0, 0)),
            pl.BlockSpec((G * E, G), lambda i: (0, 0)),
            pl.BlockSpec((1, G * 32), lambda i: (0, 0)),
        ],
        out_specs=pl.BlockSpec((BT, 1), lambda i: (i, 0)),
        out_shape=jax.ShapeDtypeStruct((B, 1), jnp.float32),
    )(fm_flat, lin, wbig, hbig, qbig, bias)


def kernel(indices, fm_table, linear_table, att_W, att_b, att_h, proj_p,
           training):
    del training
    offsets = (jnp.arange(F, dtype=indices.dtype) * (V + 1))[None, :]
    idx = (indices + offsets).astype(jnp.int32)

    fm_rows = jnp.take(fm_table, idx.reshape(-1), axis=0)
    lin_vals = jnp.take(linear_table, idx.reshape(-1), axis=0)

    wc = jnp.zeros((E, 32), jnp.float32).at[:, :T].set(att_W)
    eye = jnp.eye(G, dtype=jnp.float32)
    wbig = jnp.kron(eye, wc)
    hpad = jnp.zeros((32, 1), jnp.float32).at[:T, 0].set(att_h[:, 0])
    hbig = jnp.kron(eye, hpad)
    qbig = jnp.kron(eye, proj_p)
    bias = jnp.tile(
        jnp.concatenate([att_b, jnp.zeros((T,), jnp.float32)])[None, :],
        (1, G))

    fm_flat = fm_rows.reshape(B, F * E)
    lin = lin_vals.reshape(B, F)
    return _afm_stage(fm_flat, lin, wbig, hbig, qbig, bias)
